# baseline (speedup 1.0000x reference)
import functools

import jax
import jax.numpy as jnp
from jax import lax
from jax.experimental import pallas as pl
from jax.experimental.pallas import tpu as pltpu
from jax.experimental.pallas import tpu_sc as plsc

VOCAB = 1000000
DIM = 32
SEQ = 50
BATCH = 4096
NC = 2
NS = 16
LANES = 16
CPT = 2 * SEQ
W = 8


def _plane_body(qT_hbm, tableT_hbm, outT_hbm, qblk_v, vals_v, zv_v, shared, qsem, gsem, osem):
    cid = lax.axis_index("c")
    sid = lax.axis_index("s")

    def chunk_sb(i):
        return i // 2, sid * 256 + (i % 2) * 128

    def q_copy(i):
        s, b = chunk_sb(i)
        return pltpu.make_async_copy(
            qT_hbm.at[s, pl.ds(b, 128)], qblk_v.at[pl.ds(i * 128, 128)], qsem
        )

    def run():
        def qfire(i, c):
            q_copy(i).start()
            return c

        def qdrain(i, c):
            q_copy(i).wait()
            return c

        lax.fori_loop(0, CPT, qfire, 0)
        lax.fori_loop(0, CPT, qdrain, 0)

        zv_v[pl.ds(0, LANES)] = jnp.zeros((LANES,), jnp.float32)

        def remap(g, c):
            v = qblk_v[pl.ds(g * LANES, LANES)]
            qblk_v[pl.ds(g * LANES, LANES)] = jnp.where(v == 0, 0, v + 8)
            return c

        lax.fori_loop(0, CPT * 128 // LANES, remap, 0)

        @pl.when(sid == 0)
        def _():
            pltpu.sync_copy(zv_v.at[pl.ds(0, 8)], shared.at[pl.ds(0, 8)])

        def gather(i):
            return pltpu.make_async_copy(
                shared.at[qblk_v.at[pl.ds(i * 128, 128)]],
                vals_v.at[pl.ds(i * 128, 128)],
                gsem,
            )

        def plane_body(p, carry):
            d = cid * NS + p

            @pl.when(sid == 0)
            def _():
                pltpu.sync_copy(tableT_hbm.at[d], shared.at[pl.ds(8, VOCAB)])

            plsc.subcore_barrier()

            def store(i):
                s, b = chunk_sb(i)
                return pltpu.make_async_copy(
                    vals_v.at[pl.ds(i * 128, 128)],
                    outT_hbm.at[s, d, pl.ds(b, 128)],
                    osem,
                )

            def fire(i, c):
                gather(i).start()
                return c

            def serve(i, c):
                gather(i).wait()
                store(i).start()
                gather(i + W).start()
                return c

            def tail(i, c):
                gather(i).wait()
                store(i).start()
                return c

            def drain(i, c):
                store(i).wait()
                return c

            lax.fori_loop(0, W, fire, 0)
            lax.fori_loop(0, CPT - W, serve, 0)
            lax.fori_loop(CPT - W, CPT, tail, 0)
            lax.fori_loop(0, CPT, drain, 0)
            plsc.subcore_barrier()
            return carry

        lax.fori_loop(0, NS, plane_body, 0)

    run()


@jax.jit
def _plane_gather(tableT, qT):
    mesh = plsc.VectorSubcoreMesh(core_axis_name="c", subcore_axis_name="s")
    k = functools.partial(
        pl.kernel,
        mesh=mesh,
        out_type=jax.ShapeDtypeStruct((SEQ, DIM, BATCH), jnp.float32),
        compiler_params=pltpu.CompilerParams(
            needs_layout_passes=False, use_tc_tiling_on_sc=False
        ),
        scratch_types=[
            pltpu.VMEM((CPT * 128,), jnp.int32),
            pltpu.VMEM((CPT * 128,), jnp.float32),
            pltpu.VMEM((LANES,), jnp.float32),
            pltpu.VMEM_SHARED((VOCAB + 8,), jnp.float32),
            pltpu.SemaphoreType.DMA,
            pltpu.SemaphoreType.DMA,
            pltpu.SemaphoreType.DMA,
        ],
    )(_plane_body)
    return k(qT, tableT)


def kernel(q, q_len, table):
    outT = _plane_gather(table.T, q.T)
    return outT.transpose(2, 0, 1)


# speedup vs baseline: 8.9605x; 8.9605x over previous
import functools

import jax
import jax.numpy as jnp
from jax import lax
from jax.experimental import pallas as pl
from jax.experimental.pallas import tpu as pltpu
from jax.experimental.pallas import tpu_sc as plsc

VOCAB = 1000000
DIM = 32
SEQ = 50
BATCH = 4096
NC = 2
NS = 16
LANES = 16
CPT = 2 * SEQ
MAIN = 999424
PADW = 1000448
KCH = 4096
NCHK = MAIN // KCH


def _det_body(tT_hbm, tail_hbm, outf_hbm, stg_v, ssem, wsem):
    cid = lax.axis_index("c")
    sid = lax.axis_index("s")

    def do_chunk(g, c0, ch, src):
        pltpu.make_async_copy(
            src, stg_v.at[pl.ds(0, 8), pl.ds(0, ch)], ssem
        ).start()
        pltpu.make_async_copy(
            src, stg_v.at[pl.ds(0, 8), pl.ds(0, ch)], ssem
        ).wait()

        def wr(k, c):
            dl = k // (ch // 1024)
            j = k % (ch // 1024)
            pltpu.make_async_copy(
                stg_v.at[dl, pl.ds(j * 1024, 1024)],
                outf_hbm.at[pl.ds((8 * g + dl) * PADW + c0 + j * 1024, 1024)],
                wsem,
            ).start()
            return c

        def wdrain(k, c):
            dl = k // (ch // 1024)
            j = k % (ch // 1024)
            pltpu.make_async_copy(
                stg_v.at[dl, pl.ds(j * 1024, 1024)],
                outf_hbm.at[pl.ds((8 * g + dl) * PADW + c0 + j * 1024, 1024)],
                wsem,
            ).wait()
            return c

        nw = 8 * (ch // 1024)
        lax.fori_loop(0, nw, wr, 0)
        lax.fori_loop(0, nw, wdrain, 0)

    for gg in range(2):
        g = cid * 2 + gg

        def task(t, c, g=g):
            @pl.when(t % NS == sid)
            def _():
                c0 = t * KCH
                do_chunk(g, c0, KCH, tT_hbm.at[pl.ds(8 * g, 8), pl.ds(c0, KCH)])

            return c

        lax.fori_loop(0, NCHK, task, 0)

        @pl.when(sid == NS - 1)
        def _(g=g):
            do_chunk(g, MAIN, 1024, tail_hbm.at[pl.ds(8 * g, 8), pl.ds(0, 1024)])


@jax.jit
def _detile(tT, tail):
    mesh = plsc.VectorSubcoreMesh(core_axis_name="c", subcore_axis_name="s")
    k = functools.partial(
        pl.kernel,
        mesh=mesh,
        out_type=jax.ShapeDtypeStruct((DIM * PADW,), jnp.float32),
        compiler_params=pltpu.CompilerParams(
            needs_layout_passes=False, use_tc_tiling_on_sc=True
        ),
        scratch_types=[
            pltpu.VMEM((8, KCH), jnp.float32),
            pltpu.SemaphoreType.DMA,
            pltpu.SemaphoreType.DMA,
        ],
    )(_det_body)
    return k(tT, tail)


def _plane_body(qT_hbm, tpad_hbm, out_hbm, qblk_v, vals_v, zv_v, shared, qsem, gsem, osem):
    cid = lax.axis_index("c")
    sid = lax.axis_index("s")

    def chunk_sb(i):
        return i // 2, sid * 256 + (i % 2) * 128

    def q_copy(i):
        s, b = chunk_sb(i)
        return pltpu.make_async_copy(
            qT_hbm.at[s, pl.ds(b, 128)], qblk_v.at[pl.ds(i * 128, 128)], qsem
        )

    def qfire(i, c):
        q_copy(i).start()
        return c

    def qdrain(i, c):
        q_copy(i).wait()
        return c

    lax.fori_loop(0, CPT, qfire, 0)
    lax.fori_loop(0, CPT, qdrain, 0)

    zv_v[pl.ds(0, LANES)] = jnp.zeros((LANES,), jnp.float32)

    def remap(g, c):
        v = qblk_v[pl.ds(g * LANES, LANES)]
        qblk_v[pl.ds(g * LANES, LANES)] = jnp.where(v == 0, 0, v + 8)
        return c

    lax.fori_loop(0, CPT * 128 // LANES, remap, 0)

    @pl.when(sid == 0)
    def _():
        pltpu.sync_copy(zv_v.at[pl.ds(0, 8)], shared.at[pl.ds(0, 8)])

    def gather(i):
        return pltpu.make_async_copy(
            shared.at[qblk_v.at[pl.ds(i * 128, 128)]],
            vals_v.at[pl.ds(i * 128, 128)],
            gsem,
        )

    def stage(p):
        pltpu.sync_copy(
            tpad_hbm.at[cid * NS + p, pl.ds(0, VOCAB)], shared.at[pl.ds(8, VOCAB)]
        )

    @pl.when(sid == 0)
    def _():
        stage(0)

    def plane_body(p, carry):
        d = cid * NS + p
        plsc.subcore_barrier()

        def store(i):
            s, b = chunk_sb(i)
            return pltpu.make_async_copy(
                vals_v.at[pl.ds(i * 128, 128)],
                out_hbm.at[s, d // 8, b // 128, d % 8],
                osem,
            )

        def gfire(i, c):
            gather(i).start()
            return c

        def gdrain(i, c):
            gather(i).wait()
            return c

        def sfire(i, c):
            store(i).start()
            return c

        def sdrain(i, c):
            store(i).wait()
            return c

        lax.fori_loop(0, CPT, gfire, 0)
        lax.fori_loop(0, CPT, gdrain, 0)
        plsc.subcore_barrier()

        @pl.when((sid == 0) & (p + 1 < NS))
        def _():
            stage(p + 1)

        lax.fori_loop(0, CPT, sfire, 0)
        lax.fori_loop(0, CPT, sdrain, 0)
        return carry

    lax.fori_loop(0, NS, plane_body, 0)


@jax.jit
def _plane_gather(tpad, qT):
    mesh = plsc.VectorSubcoreMesh(core_axis_name="c", subcore_axis_name="s")
    k = functools.partial(
        pl.kernel,
        mesh=mesh,
        out_type=jax.ShapeDtypeStruct((SEQ, DIM // 8, BATCH // 128, 8, 128), jnp.float32),
        compiler_params=pltpu.CompilerParams(
            needs_layout_passes=False, use_tc_tiling_on_sc=False
        ),
        scratch_types=[
            pltpu.VMEM((CPT * 128,), jnp.int32),
            pltpu.VMEM((CPT * 128,), jnp.float32),
            pltpu.VMEM((LANES,), jnp.float32),
            pltpu.VMEM_SHARED((VOCAB + 8,), jnp.float32),
            pltpu.SemaphoreType.DMA,
            pltpu.SemaphoreType.DMA,
            pltpu.SemaphoreType.DMA,
        ],
    )(_plane_body)
    return k(qT, tpad)


def kernel(q, q_len, table):
    tT = table.T
    tail = jnp.pad(tT[:, MAIN:], ((0, 0), (0, PADW - VOCAB)))
    tpad = _detile(tT, tail).reshape(DIM, PADW)
    out5 = _plane_gather(tpad, q.T)
    return out5.transpose(2, 4, 0, 1, 3).reshape(BATCH, SEQ, DIM)


# speedup vs baseline: 9.6217x; 1.0738x over previous
import functools

import jax
import jax.numpy as jnp
from jax import lax
from jax.experimental import pallas as pl
from jax.experimental.pallas import tpu as pltpu
from jax.experimental.pallas import tpu_sc as plsc

VOCAB = 1000000
DIM = 32
SEQ = 50
BATCH = 4096
NC = 2
NS = 16
LANES = 16
CPT = 2 * SEQ
MAIN = 999424
PADW = 1000448
KCH = 4096
NCHK = MAIN // KCH


def _det_body(tT_hbm, tail_hbm, outf_hbm, stg_v, ssem0, ssem1, wsem):
    cid = lax.axis_index("c")
    sid = lax.axis_index("s")
    ssems = (ssem0, ssem1)
    nmine = (NCHK - 1 - sid) // NS + 1

    def my_t(m):
        return m * NS + sid

    def run_group(g):
        def stage(m, b):
            c0 = my_t(m) * KCH
            return pltpu.make_async_copy(
                tT_hbm.at[pl.ds(8 * g, 8), pl.ds(c0, KCH)],
                stg_v.at[b, pl.ds(0, 8), pl.ds(0, KCH)],
                ssems[b],
            )

        def wr(m, b, k):
            c0 = my_t(m) * KCH
            dl = k // (KCH // 1024)
            j = k % (KCH // 1024)
            return pltpu.make_async_copy(
                stg_v.at[b, dl, pl.ds(j * 1024, 1024)],
                outf_hbm.at[pl.ds((8 * g + dl) * PADW + c0 + j * 1024, 1024)],
                wsem,
            )

        nw = 8 * (KCH // 1024)

        stage(0, 0).start()

        def task_pair(mp, c):
            for b in range(2):
                m = mp * 2 + b

                @pl.when(m < nmine)
                def _(m=m, b=b):
                    stage(m, b).wait()

                    @pl.when(m + 1 < nmine)
                    def _():
                        stage(m + 1, 1 - b).start()

                    def wfire(k, c2):
                        wr(m, b, k).start()
                        return c2

                    def wdrain(k, c2):
                        wr(m, b, k).wait()
                        return c2

                    lax.fori_loop(0, nw, wfire, 0)
                    lax.fori_loop(0, nw, wdrain, 0)

            return c

        lax.fori_loop(0, (nmine + 1) // 2, task_pair, 0)

        @pl.when(sid == NS - 1)
        def _():
            def tstage():
                return pltpu.make_async_copy(
                    tail_hbm.at[pl.ds(8 * g, 8), pl.ds(0, 1024)],
                    stg_v.at[0, pl.ds(0, 8), pl.ds(0, 1024)],
                    ssem0,
                )

            tstage().start()
            tstage().wait()

            def twr(k):
                return pltpu.make_async_copy(
                    stg_v.at[0, k, pl.ds(0, 1024)],
                    outf_hbm.at[pl.ds((8 * g + k) * PADW + MAIN, 1024)],
                    wsem,
                )

            def twfire(k, c):
                twr(k).start()
                return c

            def twdrain(k, c):
                twr(k).wait()
                return c

            lax.fori_loop(0, 8, twfire, 0)
            lax.fori_loop(0, 8, twdrain, 0)

    for gg in range(2):
        run_group(cid * 2 + gg)


@jax.jit
def _detile(tT, tail):
    mesh = plsc.VectorSubcoreMesh(core_axis_name="c", subcore_axis_name="s")
    k = functools.partial(
        pl.kernel,
        mesh=mesh,
        out_type=jax.ShapeDtypeStruct((DIM * PADW,), jnp.float32),
        compiler_params=pltpu.CompilerParams(
            needs_layout_passes=False, use_tc_tiling_on_sc=True
        ),
        scratch_types=[
            pltpu.VMEM((2, 8, KCH), jnp.float32),
            pltpu.SemaphoreType.DMA,
            pltpu.SemaphoreType.DMA,
            pltpu.SemaphoreType.DMA,
        ],
    )(_det_body)
    return k(tT, tail)


def _plane_body(qT_hbm, tpad_hbm, out_hbm, qblk_v, vals_v, zv_v, shared, qsem, gsem, osem):
    cid = lax.axis_index("c")
    sid = lax.axis_index("s")

    def chunk_sb(i):
        return i // 2, sid * 256 + (i % 2) * 128

    def q_copy(i):
        s, b = chunk_sb(i)
        return pltpu.make_async_copy(
            qT_hbm.at[s, pl.ds(b, 128)], qblk_v.at[pl.ds(i * 128, 128)], qsem
        )

    def qfire(i, c):
        q_copy(i).start()
        return c

    def qdrain(i, c):
        q_copy(i).wait()
        return c

    lax.fori_loop(0, CPT, qfire, 0)
    lax.fori_loop(0, CPT, qdrain, 0)

    zv_v[pl.ds(0, LANES)] = jnp.zeros((LANES,), jnp.float32)

    def remap(g, c):
        v = qblk_v[pl.ds(g * LANES, LANES)]
        qblk_v[pl.ds(g * LANES, LANES)] = jnp.where(v == 0, 0, v + 8)
        return c

    lax.fori_loop(0, CPT * 128 // LANES, remap, 0)

    @pl.when(sid == 0)
    def _():
        pltpu.sync_copy(zv_v.at[pl.ds(0, 8)], shared.at[pl.ds(0, 8)])

    def gather(i):
        return pltpu.make_async_copy(
            shared.at[qblk_v.at[pl.ds(i * 128, 128)]],
            vals_v.at[pl.ds(i * 128, 128)],
            gsem,
        )

    def stage(p):
        pltpu.sync_copy(
            tpad_hbm.at[cid * NS + p, pl.ds(0, VOCAB)], shared.at[pl.ds(8, VOCAB)]
        )

    @pl.when(sid == 0)
    def _():
        stage(0)

    def plane_body(p, carry):
        d = cid * NS + p
        plsc.subcore_barrier()

        def store(i):
            s, b = chunk_sb(i)
            return pltpu.make_async_copy(
                vals_v.at[pl.ds(i * 128, 128)],
                out_hbm.at[s, d // 8, b // 128, d % 8],
                osem,
            )

        def gfire(i, c):
            gather(i).start()
            return c

        def gdrain(i, c):
            gather(i).wait()
            return c

        def sfire(i, c):
            store(i).start()
            return c

        def sdrain(i, c):
            store(i).wait()
            return c

        lax.fori_loop(0, CPT, gfire, 0)
        lax.fori_loop(0, CPT, gdrain, 0)
        plsc.subcore_barrier()

        @pl.when((sid == 0) & (p + 1 < NS))
        def _():
            stage(p + 1)

        lax.fori_loop(0, CPT, sfire, 0)
        lax.fori_loop(0, CPT, sdrain, 0)
        return carry

    lax.fori_loop(0, NS, plane_body, 0)


@jax.jit
def _plane_gather(tpad, qT):
    mesh = plsc.VectorSubcoreMesh(core_axis_name="c", subcore_axis_name="s")
    k = functools.partial(
        pl.kernel,
        mesh=mesh,
        out_type=jax.ShapeDtypeStruct((SEQ, DIM // 8, BATCH // 128, 8, 128), jnp.float32),
        compiler_params=pltpu.CompilerParams(
            needs_layout_passes=False, use_tc_tiling_on_sc=False
        ),
        scratch_types=[
            pltpu.VMEM((CPT * 128,), jnp.int32),
            pltpu.VMEM((CPT * 128,), jnp.float32),
            pltpu.VMEM((LANES,), jnp.float32),
            pltpu.VMEM_SHARED((VOCAB + 8,), jnp.float32),
            pltpu.SemaphoreType.DMA,
            pltpu.SemaphoreType.DMA,
            pltpu.SemaphoreType.DMA,
        ],
    )(_plane_body)
    return k(qT, tpad)


def kernel(q, q_len, table):
    tT = table.T
    tail = jnp.pad(tT[:, MAIN:], ((0, 0), (0, PADW - VOCAB)))
    tpad = _detile(tT, tail).reshape(DIM, PADW)
    out5 = _plane_gather(tpad, q.T)
    return out5.transpose(2, 4, 0, 1, 3).reshape(BATCH, SEQ, DIM)
